# baseline (speedup 1.0000x reference)
import jax
import jax.numpy as jnp
from jax import lax
from jax.experimental import pallas as pl
from jax.experimental.pallas import tpu as pltpu
from jax.experimental.pallas import tpu_sc as plsc

N = 10000
E = 320000
D = 128
H = 128
C = 5
N_MASK = 3000
RB = 1000
NB = N // RB
G = 128
EROWS = E // G
TILES = 16
ROWS_PER_TILE = 160
EROWS_PAD = ROWS_PER_TILE * TILES
ER2 = 2 * EROWS_PAD
ERTOT = ER2 + ROWS_PER_TILE
HROWS = EROWS_PAD // 2
HPT = HROWS // TILES
HRANGE = 5000
NH = 5120
ZCHUNK = NH // 8
DZCHUNK = NH // TILES
CHUNK = 1
CG = CHUNK * G
ER5 = ERTOT // CHUNK
NCH0 = ROWS_PER_TILE // CHUNK
NCH1 = HPT // CHUNK
OCHUNK = 1000
DOCHUNK = 1000
DSPREAD = 64


def _prep_body(x_t_ref, x_s_ref, tok_ref, w_e2d_ref, w_dec_ref, w_cls_ref,
               xm_ref, w_ed_ref, w_ec_ref):
    i = pl.program_id(0)
    half = i % 2
    j = i // 2

    @pl.when(j < N_MASK // RB)
    def _():
        xm_ref[...] = jnp.broadcast_to(tok_ref[...], (RB, D))

    @pl.when(j >= N_MASK // RB)
    def _():
        xm_ref[...] = jnp.where(half == 0, x_t_ref[...], x_s_ref[...])

    @pl.when(i == 0)
    def _():
        w = w_e2d_ref[...]
        w_ed_ref[...] = jnp.dot(w, w_dec_ref[...],
                                preferred_element_type=jnp.float32)
        w_ec_ref[...] = jnp.dot(w, w_cls_ref[...],
                                preferred_element_type=jnp.float32)


def _prep(x_t, x_s, tok, w_e2d, w_dec, w_cls_pad):
    blk_b = pl.BlockSpec((RB, D), lambda i: (i // 2, 0))
    xm_blk = pl.BlockSpec((RB, D), lambda i: ((i % 2) * NB + i // 2, 0))
    const = pl.BlockSpec((H, D), lambda i: (0, 0))
    return pl.pallas_call(
        _prep_body,
        grid=(2 * NB,),
        in_specs=[blk_b, blk_b, pl.BlockSpec((1, D), lambda i: (0, 0)),
                  const, const, const],
        out_specs=[xm_blk, const, const],
        out_shape=[
            jax.ShapeDtypeStruct((2 * N, D), jnp.float32),
            jax.ShapeDtypeStruct((H, D), jnp.float32),
            jax.ShapeDtypeStruct((H, D), jnp.float32),
        ],
    )(x_t, x_s, tok, w_e2d, w_dec, w_cls_pad)


def _sc_body(xm, esrc5, edst5, zfeat, ones_h,
             agg_out, deg_out,
             src_idx5, dst_rel5, rows0, rows1, ones_v, deg_v,
             agg_sh, deg_sh, semg0, semg1, sems0, sems1, semd):
    c = lax.axis_index("c")
    s = lax.axis_index("s")

    pltpu.sync_copy(ones_h, ones_v)

    def pass_body(p):
        @pl.when(s < 8)
        def _():
            pltpu.sync_copy(zfeat, agg_sh.at[pl.ds(s * ZCHUNK, ZCHUNK), :])

        def zbody(j, zc):
            deg_v[pl.ds(j * 16, 16)] = jnp.zeros((16,), jnp.float32)
            return zc

        lax.fori_loop(0, DZCHUNK // 16, zbody, 0)
        pltpu.sync_copy(deg_v.at[pl.ds(0, DZCHUNK)],
                        deg_sh.at[pl.ds(s * DZCHUNK, DZCHUNK)])

        base50 = c * (EROWS_PAD // CHUNK) + s * NCH0
        base51 = (EROWS_PAD // CHUNK) + c * (HROWS // CHUNK) + s * NCH1
        base5 = jnp.where(p == 0, base50, base51)
        base5 = pl.multiple_of(base5, 8)
        dbase5 = p * ER5 + base5
        dbase5 = pl.multiple_of(dbase5, 8)
        nch = jnp.where(p == 0, NCH0, NCH1)
        slot = 2 * p + c

        pltpu.sync_copy(esrc5.at[pl.ds(base5, NCH0), :], src_idx5)
        pltpu.sync_copy(edst5.at[pl.ds(dbase5, NCH0), :], dst_rel5)
        plsc.subcore_barrier()

        pltpu.async_copy(xm.at[src_idx5.at[0]], rows0, semg0)

        def pair(q, pc):
            for k in (0, 1):
                ch = 2 * q + k
                rows_k = rows0 if k == 0 else rows1
                rows_o = rows1 if k == 0 else rows0
                semg_k = semg0 if k == 0 else semg1
                semg_o = semg1 if k == 0 else semg0
                pltpu.make_async_copy(
                    xm.at[src_idx5.at[0]], rows_k, semg_k).wait()

                @pl.when(ch >= 1)
                def _():
                    pltpu.make_async_copy(
                        rows_o, agg_sh.at[dst_rel5.at[0]], sems0).wait()

                @pl.when(ch + 1 < nch)
                def _():
                    pltpu.async_copy(
                        xm.at[src_idx5.at[ch + 1]], rows_o, semg_o)

                pltpu.async_copy(
                    rows_k, agg_sh.at[dst_rel5.at[ch]], sems0, add=True)

                @pl.when(ch >= 1)
                def _():
                    pltpu.make_async_copy(
                        ones_v, deg_sh.at[dst_rel5.at[0]], semd).wait()

                pltpu.async_copy(
                    ones_v, deg_sh.at[dst_rel5.at[ch]], semd, add=True)
            return pc

        lax.fori_loop(0, nch // 2, pair, 0)
        pltpu.make_async_copy(rows1, agg_sh.at[dst_rel5.at[0]], sems0).wait()
        pltpu.make_async_copy(ones_v, deg_sh.at[dst_rel5.at[0]], semd).wait()
        plsc.subcore_barrier()

        @pl.when(s < 5)
        def _():
            sl = pl.ds(s * OCHUNK, OCHUNK)
            pltpu.sync_copy(agg_sh.at[sl, :], agg_out.at[slot, sl, :])

        @pl.when((s >= 5) & (s < 10))
        def _():
            dsl = pl.ds((s - 5) * DOCHUNK, DOCHUNK)
            pltpu.sync_copy(deg_sh.at[dsl], deg_v.at[pl.ds(0, DOCHUNK)])
            doff = slot * HRANGE + (s - 5) * DOCHUNK
            pltpu.sync_copy(deg_v.at[pl.ds(0, DOCHUNK)],
                            deg_out.at[pl.ds(doff, DOCHUNK)])

        plsc.subcore_barrier()

    for p in (0, 1):
        pass_body(jnp.int32(p))


def _sc_agg(xm, esrc5, edst5, zfeat, ones_h):
    mesh = plsc.VectorSubcoreMesh(core_axis_name="c", subcore_axis_name="s",
                                  num_cores=2, num_subcores=TILES)
    f32 = jnp.float32
    kern = pl.kernel(
        _sc_body,
        out_type=[
            jax.ShapeDtypeStruct((4, HRANGE, D), f32),
            jax.ShapeDtypeStruct((4 * HRANGE,), f32),
        ],
        mesh=mesh,
        scratch_types=[
            pltpu.VMEM((NCH0, CG), jnp.int32),
            pltpu.VMEM((NCH0, CG), jnp.int32),
            pltpu.VMEM((CG, D), f32),
            pltpu.VMEM((CG, D), f32),
            pltpu.VMEM((CG,), f32),
            pltpu.VMEM((DOCHUNK,), f32),
            pltpu.VMEM_SHARED((NH, D), f32),
            pltpu.VMEM_SHARED((NH,), f32),
            pltpu.SemaphoreType.DMA,
            pltpu.SemaphoreType.DMA,
            pltpu.SemaphoreType.DMA,
            pltpu.SemaphoreType.DMA,
            pltpu.SemaphoreType.DMA,
        ],
    )
    return kern(xm, esrc5, edst5, zfeat, ones_h)


def _sce_rows(recon, x):
    rn = recon / (jnp.sqrt(jnp.sum(recon * recon, axis=1, keepdims=True)) + 1e-8)
    xn = x / (jnp.sqrt(jnp.sum(x * x, axis=1, keepdims=True)) + 1e-8)
    cos = jnp.sum(rn * xn, axis=1, keepdims=True)
    one_m = 1.0 - cos
    return jnp.sum(one_m * one_m)


def _finale_body(agg_t_ref, deg_t_ref, agg_s0_ref, deg_s0_ref,
                 agg_s1a_ref, deg_s1a_ref, agg_s1b_ref, deg_s1b_ref,
                 x_t_ref, x_s_ref, y_ref,
                 w_ed_ref, b_dec_ref, w_ec_ref, b_cls_ref, out_ref):
    i = pl.program_id(0)

    @pl.when(i == 0)
    def _():
        out_ref[...] = jnp.zeros((1, 1), jnp.float32)

    in_lo = i < HRANGE // RB
    agg_s = jnp.where(in_lo, agg_s0_ref[...],
                      agg_s1a_ref[...] + agg_s1b_ref[...])
    deg_s = jnp.where(in_lo, deg_s0_ref[...],
                      deg_s1a_ref[...] + deg_s1b_ref[...])
    h_s = agg_s / jnp.maximum(deg_s, 1.0)
    logits = jnp.dot(h_s, w_ec_ref[...],
                     preferred_element_type=jnp.float32) + b_cls_ref[...]
    col = lax.broadcasted_iota(jnp.int32, (RB, D), 1)
    valid = col < C
    lm = jnp.where(valid, logits, jnp.float32(-1e30))
    mx = jnp.max(lm, axis=1, keepdims=True)
    ex = jnp.where(valid, jnp.exp(lm - mx), 0.0)
    lse = jnp.log(jnp.sum(ex, axis=1, keepdims=True)) + mx
    sel = col == y_ref[...]
    logit_y = jnp.sum(jnp.where(sel, lm, 0.0), axis=1, keepdims=True)
    ce_sum = jnp.sum(lse - logit_y)

    h_t = agg_t_ref[...] / jnp.maximum(deg_t_ref[...], 1.0)
    recon_t = jnp.dot(h_t, w_ed_ref[...],
                      preferred_element_type=jnp.float32) + b_dec_ref[...]
    recon_s = jnp.dot(h_s, w_ed_ref[...],
                      preferred_element_type=jnp.float32) + b_dec_ref[...]
    sce_blk = _sce_rows(recon_t, x_t_ref[...]) + _sce_rows(recon_s, x_s_ref[...])
    flag = jnp.where(i < N_MASK // RB, 1.0, 0.0)
    contrib = ce_sum / N + flag * sce_blk / N_MASK
    out_ref[...] += contrib.reshape(1, 1)


def _finale(agg_t, deg_t, agg_s0, deg_s0, agg_s1a, deg_s1a, agg_s1b, deg_s1b,
            x_t, x_s, y2d, w_ed, b_dec2, w_ec, b_cls2):
    mcap = N_MASK // RB - 1
    locap = HRANGE // RB - 1
    blk = pl.BlockSpec((RB, D), lambda i: (i, 0))
    blk_m = pl.BlockSpec((RB, D), lambda i: (jnp.minimum(i, mcap), 0))
    blk_lo = pl.BlockSpec((RB, D), lambda i: (jnp.minimum(i, locap), 0))
    blk_hi = pl.BlockSpec((RB, D),
                          lambda i: (jnp.maximum(i - (locap + 1), 0), 0))
    col1_m = pl.BlockSpec((RB, 1), lambda i: (jnp.minimum(i, mcap), 0))
    col1_lo = pl.BlockSpec((RB, 1), lambda i: (jnp.minimum(i, locap), 0))
    col1_hi = pl.BlockSpec((RB, 1),
                           lambda i: (jnp.maximum(i - (locap + 1), 0), 0))
    col1 = pl.BlockSpec((RB, 1), lambda i: (i, 0))
    const = pl.BlockSpec((H, D), lambda i: (0, 0))
    row = pl.BlockSpec((1, D), lambda i: (0, 0))
    out = pl.pallas_call(
        _finale_body,
        grid=(NB,),
        in_specs=[blk_m, col1_m, blk_lo, col1_lo,
                  blk_hi, col1_hi, blk_hi, col1_hi,
                  blk_m, blk, col1, const, row, const, row],
        out_specs=pl.BlockSpec((1, 1), lambda i: (0, 0)),
        out_shape=jax.ShapeDtypeStruct((1, 1), jnp.float32),
    )(agg_t, deg_t, agg_s0, deg_s0, agg_s1a, deg_s1a, agg_s1b, deg_s1b,
      x_t, x_s, y2d, w_ed, b_dec2, w_ec, b_cls2)
    return out


def kernel(x_t, edge_index_t, x_s, edge_index_s, y_s,
           enc_mask_token, W_e2d, W_dec, b_dec, W_cls, b_cls):
    f32 = jnp.float32
    w_cls_pad = jnp.zeros((H, D), f32).at[:, :C].set(W_cls)
    b_cls_pad = jnp.zeros((1, D), f32).at[0, :C].set(b_cls)
    b_dec2 = b_dec.reshape(1, D)

    xm, w_ed, w_ec = _prep(x_t, x_s, enc_mask_token, W_e2d, W_dec, w_cls_pad)

    def pack_edges(ei):
        src = ei[0].reshape(EROWS, G)
        dst = ei[1].reshape(EROWS, G)
        pad_n = EROWS_PAD - EROWS
        src = jnp.concatenate(
            [src, jnp.zeros((pad_n, G), jnp.int32)], axis=0)
        dst = jnp.concatenate(
            [dst, jnp.full((pad_n, G), N, jnp.int32)], axis=0)
        return src, dst

    src_t, dst_t = pack_edges(edge_index_t)
    src_s, dst_s = pack_edges(edge_index_s)
    esrc = jnp.concatenate(
        [src_t, src_s + N, jnp.zeros((ERTOT - ER2, G), jnp.int32)],
        axis=0)
    edst = jnp.concatenate(
        [dst_t, dst_s, jnp.full((ERTOT - ER2, G), N, jnp.int32)],
        axis=0)
    dummy = HRANGE + (
        jnp.arange(ERTOT * G, dtype=jnp.int32).reshape(ERTOT, G) % DSPREAD)
    dstrel0 = jnp.where(edst < HRANGE, edst, dummy).astype(jnp.int32)
    r1 = edst - HRANGE
    dstrel1 = jnp.where((r1 >= 0) & (r1 < HRANGE), r1, dummy).astype(jnp.int32)
    esrc5 = esrc.reshape(ER5, CG)
    edst5 = jnp.concatenate([dstrel0, dstrel1], axis=0).reshape(2 * ER5, CG)
    zfeat = jnp.zeros((ZCHUNK, D), f32)
    ones_h = jnp.ones((CG,), f32)

    agg_all, deg_all = _sc_agg(xm, esrc5, edst5, zfeat, ones_h)
    deg3 = deg_all.reshape(4, HRANGE, 1)

    y2d = y_s.reshape(N, 1)
    out = _finale(agg_all[0], deg3[0], agg_all[1], deg3[1],
                  agg_all[2], deg3[2], agg_all[3], deg3[3],
                  x_t, x_s, y2d, w_ed, b_dec2, w_ec, b_cls_pad)
    return out[0, 0]


# speedup vs baseline: 1.0018x; 1.0018x over previous
import jax
import jax.numpy as jnp
from jax import lax
from jax.experimental import pallas as pl
from jax.experimental.pallas import tpu as pltpu
from jax.experimental.pallas import tpu_sc as plsc

N = 10000
E = 320000
D = 128
H = 128
C = 5
N_MASK = 3000
RB = 1000
NB = N // RB
G = 128
EROWS = E // G
TILES = 16
ROWS_PER_TILE = 160
EROWS_PAD = ROWS_PER_TILE * TILES
ER2 = 2 * EROWS_PAD
ERTOT = ER2 + ROWS_PER_TILE
HROWS = EROWS_PAD // 2
HPT = HROWS // TILES
HRANGE = 5000
NH = 5120
ZCHUNK = NH // 8
DZCHUNK = NH // TILES
CHUNK = 1
CG = CHUNK * G
ER5 = ERTOT // CHUNK
NCH0 = ROWS_PER_TILE // CHUNK
NCH1 = HPT // CHUNK
OCHUNK = 1000
DOCHUNK = 1000
DSPREAD = 64


def _prep_body(x_t_ref, x_s_ref, tok_ref, w_e2d_ref, w_dec_ref, w_cls_ref,
               xm_ref, w_ed_ref, w_ec_ref):
    i = pl.program_id(0)
    half = i % 2
    j = i // 2

    @pl.when(j < N_MASK // RB)
    def _():
        xm_ref[...] = jnp.broadcast_to(tok_ref[...], (RB, D))

    @pl.when(j >= N_MASK // RB)
    def _():
        xm_ref[...] = jnp.where(half == 0, x_t_ref[...], x_s_ref[...])

    @pl.when(i == 0)
    def _():
        w = w_e2d_ref[...]
        w_ed_ref[...] = jnp.dot(w, w_dec_ref[...],
                                preferred_element_type=jnp.float32)
        w_ec_ref[...] = jnp.dot(w, w_cls_ref[...],
                                preferred_element_type=jnp.float32)


def _prep(x_t, x_s, tok, w_e2d, w_dec, w_cls_pad):
    blk_b = pl.BlockSpec((RB, D), lambda i: (i // 2, 0))
    xm_blk = pl.BlockSpec((RB, D), lambda i: ((i % 2) * NB + i // 2, 0))
    const = pl.BlockSpec((H, D), lambda i: (0, 0))
    return pl.pallas_call(
        _prep_body,
        grid=(2 * NB,),
        in_specs=[blk_b, blk_b, pl.BlockSpec((1, D), lambda i: (0, 0)),
                  const, const, const],
        out_specs=[xm_blk, const, const],
        out_shape=[
            jax.ShapeDtypeStruct((2 * N, D), jnp.float32),
            jax.ShapeDtypeStruct((H, D), jnp.float32),
            jax.ShapeDtypeStruct((H, D), jnp.float32),
        ],
    )(x_t, x_s, tok, w_e2d, w_dec, w_cls_pad)


def _sc_body(xm, esrc5, edst5, zfeat, ones_h,
             agg_out, deg_out,
             src_idx5, dst_rel5, rows0, rows1, ones_v, deg_v,
             agg_sh, deg_sh, semg0, semg1, sems0, sems1, semd):
    c = lax.axis_index("c")
    s = lax.axis_index("s")

    pltpu.sync_copy(ones_h, ones_v)

    def pass_body(p):
        @pl.when(s < 8)
        def _():
            pltpu.sync_copy(zfeat, agg_sh.at[pl.ds(s * ZCHUNK, ZCHUNK), :])

        def zbody(j, zc):
            deg_v[pl.ds(j * 16, 16)] = jnp.zeros((16,), jnp.float32)
            return zc

        lax.fori_loop(0, DZCHUNK // 16, zbody, 0)
        pltpu.sync_copy(deg_v.at[pl.ds(0, DZCHUNK)],
                        deg_sh.at[pl.ds(s * DZCHUNK, DZCHUNK)])

        base50 = c * (EROWS_PAD // CHUNK) + s * NCH0
        base51 = (EROWS_PAD // CHUNK) + c * (HROWS // CHUNK) + s * NCH1
        base5 = jnp.where(p == 0, base50, base51)
        base5 = pl.multiple_of(base5, 8)
        dbase5 = p * ER5 + base5
        dbase5 = pl.multiple_of(dbase5, 8)
        nch = jnp.where(p == 0, NCH0, NCH1)
        slot = 2 * p + c

        pltpu.sync_copy(esrc5.at[pl.ds(base5, NCH0), :], src_idx5)
        pltpu.sync_copy(edst5.at[pl.ds(dbase5, NCH0), :], dst_rel5)
        plsc.subcore_barrier()

        pltpu.async_copy(xm.at[src_idx5.at[0]], rows0, semg0)

        def pair(q, pc):
            for k in (0, 1):
                ch = 2 * q + k
                rows_k = rows0 if k == 0 else rows1
                rows_o = rows1 if k == 0 else rows0
                semg_k = semg0 if k == 0 else semg1
                semg_o = semg1 if k == 0 else semg0
                pltpu.make_async_copy(
                    xm.at[src_idx5.at[0]], rows_k, semg_k).wait()

                @pl.when(ch + 1 < nch)
                def _():
                    pltpu.async_copy(
                        xm.at[src_idx5.at[ch + 1]], rows_o, semg_o)

                pltpu.sync_copy(rows_k, agg_sh.at[dst_rel5.at[ch]], add=True)

                @pl.when(ch >= 1)
                def _():
                    pltpu.make_async_copy(
                        ones_v, deg_sh.at[dst_rel5.at[0]], semd).wait()

                pltpu.async_copy(
                    ones_v, deg_sh.at[dst_rel5.at[ch]], semd, add=True)
            return pc

        lax.fori_loop(0, nch // 2, pair, 0)
        pltpu.make_async_copy(ones_v, deg_sh.at[dst_rel5.at[0]], semd).wait()
        plsc.subcore_barrier()

        @pl.when(s < 5)
        def _():
            sl = pl.ds(s * OCHUNK, OCHUNK)
            pltpu.sync_copy(agg_sh.at[sl, :], agg_out.at[slot, sl, :])

        @pl.when((s >= 5) & (s < 10))
        def _():
            dsl = pl.ds((s - 5) * DOCHUNK, DOCHUNK)
            pltpu.sync_copy(deg_sh.at[dsl], deg_v.at[pl.ds(0, DOCHUNK)])
            doff = slot * HRANGE + (s - 5) * DOCHUNK
            pltpu.sync_copy(deg_v.at[pl.ds(0, DOCHUNK)],
                            deg_out.at[pl.ds(doff, DOCHUNK)])

        plsc.subcore_barrier()

    for p in (0, 1):
        pass_body(jnp.int32(p))


def _sc_agg(xm, esrc5, edst5, zfeat, ones_h):
    mesh = plsc.VectorSubcoreMesh(core_axis_name="c", subcore_axis_name="s",
                                  num_cores=2, num_subcores=TILES)
    f32 = jnp.float32
    kern = pl.kernel(
        _sc_body,
        out_type=[
            jax.ShapeDtypeStruct((4, HRANGE, D), f32),
            jax.ShapeDtypeStruct((4 * HRANGE,), f32),
        ],
        mesh=mesh,
        scratch_types=[
            pltpu.VMEM((NCH0, CG), jnp.int32),
            pltpu.VMEM((NCH0, CG), jnp.int32),
            pltpu.VMEM((CG, D), f32),
            pltpu.VMEM((CG, D), f32),
            pltpu.VMEM((CG,), f32),
            pltpu.VMEM((DOCHUNK,), f32),
            pltpu.VMEM_SHARED((NH, D), f32),
            pltpu.VMEM_SHARED((NH,), f32),
            pltpu.SemaphoreType.DMA,
            pltpu.SemaphoreType.DMA,
            pltpu.SemaphoreType.DMA,
            pltpu.SemaphoreType.DMA,
            pltpu.SemaphoreType.DMA,
        ],
    )
    return kern(xm, esrc5, edst5, zfeat, ones_h)


def _sce_rows(recon, x):
    rn = recon / (jnp.sqrt(jnp.sum(recon * recon, axis=1, keepdims=True)) + 1e-8)
    xn = x / (jnp.sqrt(jnp.sum(x * x, axis=1, keepdims=True)) + 1e-8)
    cos = jnp.sum(rn * xn, axis=1, keepdims=True)
    one_m = 1.0 - cos
    return jnp.sum(one_m * one_m)


def _finale_body(agg_t_ref, deg_t_ref, agg_s0_ref, deg_s0_ref,
                 agg_s1a_ref, deg_s1a_ref, agg_s1b_ref, deg_s1b_ref,
                 x_t_ref, x_s_ref, y_ref,
                 w_ed_ref, b_dec_ref, w_ec_ref, b_cls_ref, out_ref):
    i = pl.program_id(0)

    @pl.when(i == 0)
    def _():
        out_ref[...] = jnp.zeros((1, 1), jnp.float32)

    in_lo = i < HRANGE // RB
    agg_s = jnp.where(in_lo, agg_s0_ref[...],
                      agg_s1a_ref[...] + agg_s1b_ref[...])
    deg_s = jnp.where(in_lo, deg_s0_ref[...],
                      deg_s1a_ref[...] + deg_s1b_ref[...])
    h_s = agg_s / jnp.maximum(deg_s, 1.0)
    logits = jnp.dot(h_s, w_ec_ref[...],
                     preferred_element_type=jnp.float32) + b_cls_ref[...]
    col = lax.broadcasted_iota(jnp.int32, (RB, D), 1)
    valid = col < C
    lm = jnp.where(valid, logits, jnp.float32(-1e30))
    mx = jnp.max(lm, axis=1, keepdims=True)
    ex = jnp.where(valid, jnp.exp(lm - mx), 0.0)
    lse = jnp.log(jnp.sum(ex, axis=1, keepdims=True)) + mx
    sel = col == y_ref[...]
    logit_y = jnp.sum(jnp.where(sel, lm, 0.0), axis=1, keepdims=True)
    ce_sum = jnp.sum(lse - logit_y)

    h_t = agg_t_ref[...] / jnp.maximum(deg_t_ref[...], 1.0)
    recon_t = jnp.dot(h_t, w_ed_ref[...],
                      preferred_element_type=jnp.float32) + b_dec_ref[...]
    recon_s = jnp.dot(h_s, w_ed_ref[...],
                      preferred_element_type=jnp.float32) + b_dec_ref[...]
    sce_blk = _sce_rows(recon_t, x_t_ref[...]) + _sce_rows(recon_s, x_s_ref[...])
    flag = jnp.where(i < N_MASK // RB, 1.0, 0.0)
    contrib = ce_sum / N + flag * sce_blk / N_MASK
    out_ref[...] += contrib.reshape(1, 1)


def _finale(agg_t, deg_t, agg_s0, deg_s0, agg_s1a, deg_s1a, agg_s1b, deg_s1b,
            x_t, x_s, y2d, w_ed, b_dec2, w_ec, b_cls2):
    mcap = N_MASK // RB - 1
    locap = HRANGE // RB - 1
    blk = pl.BlockSpec((RB, D), lambda i: (i, 0))
    blk_m = pl.BlockSpec((RB, D), lambda i: (jnp.minimum(i, mcap), 0))
    blk_lo = pl.BlockSpec((RB, D), lambda i: (jnp.minimum(i, locap), 0))
    blk_hi = pl.BlockSpec((RB, D),
                          lambda i: (jnp.maximum(i - (locap + 1), 0), 0))
    col1_m = pl.BlockSpec((RB, 1), lambda i: (jnp.minimum(i, mcap), 0))
    col1_lo = pl.BlockSpec((RB, 1), lambda i: (jnp.minimum(i, locap), 0))
    col1_hi = pl.BlockSpec((RB, 1),
                           lambda i: (jnp.maximum(i - (locap + 1), 0), 0))
    col1 = pl.BlockSpec((RB, 1), lambda i: (i, 0))
    const = pl.BlockSpec((H, D), lambda i: (0, 0))
    row = pl.BlockSpec((1, D), lambda i: (0, 0))
    out = pl.pallas_call(
        _finale_body,
        grid=(NB,),
        in_specs=[blk_m, col1_m, blk_lo, col1_lo,
                  blk_hi, col1_hi, blk_hi, col1_hi,
                  blk_m, blk, col1, const, row, const, row],
        out_specs=pl.BlockSpec((1, 1), lambda i: (0, 0)),
        out_shape=jax.ShapeDtypeStruct((1, 1), jnp.float32),
    )(agg_t, deg_t, agg_s0, deg_s0, agg_s1a, deg_s1a, agg_s1b, deg_s1b,
      x_t, x_s, y2d, w_ed, b_dec2, w_ec, b_cls2)
    return out


def kernel(x_t, edge_index_t, x_s, edge_index_s, y_s,
           enc_mask_token, W_e2d, W_dec, b_dec, W_cls, b_cls):
    f32 = jnp.float32
    w_cls_pad = jnp.zeros((H, D), f32).at[:, :C].set(W_cls)
    b_cls_pad = jnp.zeros((1, D), f32).at[0, :C].set(b_cls)
    b_dec2 = b_dec.reshape(1, D)

    xm, w_ed, w_ec = _prep(x_t, x_s, enc_mask_token, W_e2d, W_dec, w_cls_pad)

    def pack_edges(ei):
        src = ei[0].reshape(EROWS, G)
        dst = ei[1].reshape(EROWS, G)
        pad_n = EROWS_PAD - EROWS
        src = jnp.concatenate(
            [src, jnp.zeros((pad_n, G), jnp.int32)], axis=0)
        dst = jnp.concatenate(
            [dst, jnp.full((pad_n, G), N, jnp.int32)], axis=0)
        return src, dst

    src_t, dst_t = pack_edges(edge_index_t)
    src_s, dst_s = pack_edges(edge_index_s)
    esrc = jnp.concatenate(
        [src_t, src_s + N, jnp.zeros((ERTOT - ER2, G), jnp.int32)],
        axis=0)
    edst = jnp.concatenate(
        [dst_t, dst_s, jnp.full((ERTOT - ER2, G), N, jnp.int32)],
        axis=0)
    dummy = HRANGE + (
        jnp.arange(ERTOT * G, dtype=jnp.int32).reshape(ERTOT, G) % DSPREAD)
    dstrel0 = jnp.where(edst < HRANGE, edst, dummy).astype(jnp.int32)
    r1 = edst - HRANGE
    dstrel1 = jnp.where((r1 >= 0) & (r1 < HRANGE), r1, dummy).astype(jnp.int32)
    esrc5 = esrc.reshape(ER5, CG)
    edst5 = jnp.concatenate([dstrel0, dstrel1], axis=0).reshape(2 * ER5, CG)
    zfeat = jnp.zeros((ZCHUNK, D), f32)
    ones_h = jnp.ones((CG,), f32)

    agg_all, deg_all = _sc_agg(xm, esrc5, edst5, zfeat, ones_h)
    deg3 = deg_all.reshape(4, HRANGE, 1)

    y2d = y_s.reshape(N, 1)
    out = _finale(agg_all[0], deg3[0], agg_all[1], deg3[1],
                  agg_all[2], deg3[2], agg_all[3], deg3[3],
                  x_t, x_s, y2d, w_ed, b_dec2, w_ec, b_cls_pad)
    return out[0, 0]
